# speedup vs baseline: 16.2625x; 16.2625x over previous
import functools

import jax
import jax.numpy as jnp
from jax import lax
from jax.experimental import pallas as pl

NPG = 96
KNN = 16
FEA_IN = 5


def _relu(v):
    return jnp.maximum(v, 0.0)


def _leaky(v):
    return jnp.where(v >= 0, v, 0.01 * v)


def _conv_body(xx_ref, w1a1_ref, w1b1_ref, b11_ref, w21_ref, b21_ref,
               w1a2_ref, w1b2_ref, b12_ref, w22_ref, b22_ref,
               x1_ref, x2_ref, *, G):
    xx = xx_ref[...]

    d2 = jnp.zeros((G, NPG, NPG), jnp.float32)
    for c in range(3):
        pc = xx[:, :, 2 + c]
        diff = pc[:, :, None] - pc[:, None, :]
        d2 = d2 + diff * diff

    jidx = lax.broadcasted_iota(jnp.int32, (G, NPG, NPG), 2)
    work = d2
    oh_slots = []
    for _ in range(KNN):
        m = jnp.min(work, axis=-1, keepdims=True)
        eq = work == m
        selj = jnp.min(jnp.where(eq, jidx, NPG), axis=-1, keepdims=True)
        oh = jidx == selj
        oh_slots.append(oh.astype(jnp.float32))
        work = jnp.where(oh, jnp.float32(1e30), work)
    onehot = jnp.concatenate(oh_slots, axis=1)

    def edge_conv(feat, w1a_ref, w1b_ref, b1_ref, w2_ref, b2_ref, act):
        F = feat.shape[-1]
        H1 = w1a_ref.shape[-1]
        H2 = w2_ref.shape[-1]
        feat2d = feat.reshape(G * NPG, F)
        a = feat2d @ w1a_ref[...] + b1_ref[...]
        c = feat2d @ w1b_ref[...]
        cj = lax.dot_general(onehot, c.reshape(G, NPG, H1),
                             (((2,), (1,)), ((0,), (0,))))
        h = act(cj.reshape(G, KNN, NPG, H1) +
                a.reshape(G, 1, NPG, H1))
        h = act(h.reshape(G * KNN * NPG, H1) @ w2_ref[...] + b2_ref[...])
        return jnp.max(h.reshape(G, KNN, NPG, H2), axis=1)

    x1 = edge_conv(xx, w1a1_ref, w1b1_ref, b11_ref, w21_ref, b21_ref, _relu)
    x2 = edge_conv(x1, w1a2_ref, w1b2_ref, b12_ref, w22_ref, b22_ref, _leaky)
    x1_ref[...] = x1
    x2_ref[...] = x2


def _mlp_body(xx_ref, x1_ref, x2_ref,
              w1a_ref, w1b_ref, w1c_ref, b1_ref,
              w2_ref, b2_ref, w3_ref, b3_ref, w4_ref, b4_ref,
              w5_ref, b5_ref, out_ref):
    h = _relu(xx_ref[...] @ w1a_ref[...] + x1_ref[...] @ w1b_ref[...]
              + x2_ref[...] @ w1c_ref[...] + b1_ref[...])
    h = _relu(h @ w2_ref[...] + b2_ref[...])
    h = _relu(h @ w3_ref[...] + b3_ref[...])
    h = _relu(h @ w4_ref[...] + b4_ref[...])
    out_ref[...] = h @ w5_ref[...] + b5_ref[...]


def _block_diag(a, b):
    fi_a, fo_a = a.shape
    fi_b, fo_b = b.shape
    top = jnp.concatenate([a, jnp.zeros((fi_a, fo_b), a.dtype)], axis=1)
    bot = jnp.concatenate([jnp.zeros((fi_b, fo_a), b.dtype), b], axis=1)
    return jnp.concatenate([top, bot], axis=0)


@jax.jit
def kernel(x, pos, tq, batch, params):
    del batch
    N = x.shape[0]
    B = N // NPG
    p = params

    xx = jnp.concatenate([tq, x, pos], axis=1).reshape(B, NPG, FEA_IN)

    c1w1 = p['c1_w1']
    w1a1 = c1w1[:FEA_IN] - c1w1[FEA_IN:]
    w1b1 = c1w1[FEA_IN:]
    c2w1 = p['c2_w1']
    w1a2 = c2w1[:32] - c2w1[32:]
    w1b2 = c2w1[32:]

    G = 4 if B % 4 == 0 else 1
    const2 = lambda i: (0, 0)
    conv_specs = [
        pl.BlockSpec((G, NPG, FEA_IN), lambda i: (i, 0, 0)),
        pl.BlockSpec((FEA_IN, 32), const2),
        pl.BlockSpec((FEA_IN, 32), const2),
        pl.BlockSpec((1, 32), const2),
        pl.BlockSpec((32, 32), const2),
        pl.BlockSpec((1, 32), const2),
        pl.BlockSpec((32, 64), const2),
        pl.BlockSpec((32, 64), const2),
        pl.BlockSpec((1, 64), const2),
        pl.BlockSpec((64, 32), const2),
        pl.BlockSpec((1, 32), const2),
    ]
    x1, x2 = pl.pallas_call(
        functools.partial(_conv_body, G=G),
        grid=(B // G,),
        in_specs=conv_specs,
        out_specs=[pl.BlockSpec((G, NPG, 32), lambda i: (i, 0, 0))] * 2,
        out_shape=[jax.ShapeDtypeStruct((B, NPG, 32), jnp.float32)] * 2,
    )(xx, w1a1, w1b1, p['c1_b1'][None, :], p['c1_w2'], p['c1_b2'][None, :],
      w1a2, w1b2, p['c2_b1'][None, :], p['c2_w2'], p['c2_b2'][None, :])

    def split_w1(w):
        w = w.reshape(NPG, 64 + FEA_IN, w.shape[-1])
        return (w[:, :FEA_IN].reshape(NPG * FEA_IN, -1),
                w[:, FEA_IN:FEA_IN + 32].reshape(NPG * 32, -1),
                w[:, FEA_IN + 32:].reshape(NPG * 32, -1))

    ma, mb, mc = split_w1(p['m1_w'])
    na, nb, nc = split_w1(p['n1_w'])
    w1a = jnp.concatenate([ma, na], axis=1)
    w1b = jnp.concatenate([mb, nb], axis=1)
    w1c = jnp.concatenate([mc, nc], axis=1)
    b1 = jnp.concatenate([p['m1_b'], p['n1_b']])[None, :]
    w2 = _block_diag(p['m2_w'], p['n2_w'])
    b2 = jnp.concatenate([p['m2_b'], p['n2_b']])[None, :]
    w3 = _block_diag(p['m3_w'], p['n3_w'])
    b3 = jnp.concatenate([p['m3_b'], p['n3_b']])[None, :]
    w4 = _block_diag(p['m4_w'], p['n4_w'])
    b4 = jnp.concatenate([p['m4_b'], p['n4_b']])[None, :]
    w5 = jnp.zeros((128, 3), jnp.float32)
    w5 = w5.at[:64, :2].set(p['m5_w']).at[64:, 2:].set(p['n5_w'])
    b5 = jnp.concatenate([p['m5_b'], p['n5_b']])[None, :]

    BM = 128 if B % 128 == 0 else B
    mlp_specs = (
        [pl.BlockSpec((BM, NPG * FEA_IN), lambda i: (i, 0)),
         pl.BlockSpec((BM, NPG * 32), lambda i: (i, 0)),
         pl.BlockSpec((BM, NPG * 32), lambda i: (i, 0))] +
        [pl.BlockSpec(w.shape, const2) for w in
         (w1a, w1b, w1c, b1, w2, b2, w3, b3, w4, b4, w5, b5)]
    )
    out = pl.pallas_call(
        _mlp_body,
        grid=(B // BM,),
        in_specs=mlp_specs,
        out_specs=pl.BlockSpec((BM, 3), lambda i: (i, 0)),
        out_shape=jax.ShapeDtypeStruct((B, 3), jnp.float32),
    )(xx.reshape(B, NPG * FEA_IN), x1.reshape(B, NPG * 32),
      x2.reshape(B, NPG * 32),
      w1a, w1b, w1c, b1, w2, b2, w3, b3, w4, b4, w5, b5)
    return out
